# baseline (speedup 1.0000x reference)
import dataclasses
import functools

import jax
import jax.numpy as jnp
from jax import lax
from jax.experimental import pallas as pl
from jax.experimental.pallas import tpu as pltpu
from jax.experimental.pallas import tpu_sc as plsc

_BS = 32
_L = 16


@functools.partial(jax.jit, static_argnums=(1, 2, 3))
def _sc_pool(x2d, b, f, d):
    mesh = plsc.VectorSubcoreMesh(core_axis_name="core", subcore_axis_name="subcore")
    cp = pltpu.CompilerParams()
    if "needs_layout_passes" in pltpu.CompilerParams.__dataclass_fields__:
        cp = dataclasses.replace(cp, needs_layout_passes=False)

    @functools.partial(
        pl.kernel,
        out_type=jax.ShapeDtypeStruct((b // _L, _L), jnp.float32),
        mesh=mesh,
        compiler_params=cp,
    )
    def k(x_hbm, o_hbm):
        def body(x_vmem, o_vmem):
            lane = lax.iota(jnp.int32, _L)

            @pl.loop(0, _BS // _L)
            def per_group(g):
                def per_sample(j, res):
                    i = g * _L + j
                    nc = 4
                    accs = [jnp.zeros((d,), jnp.float32) for _ in range(nc)]
                    acc2s = [jnp.zeros((d,), jnp.float32) for _ in range(nc)]
                    for jf in range(f):
                        v = x_vmem[i, pl.ds(jf * d, d)]
                        c = jf % nc
                        accs[c] = accs[c] + v
                        acc2s[c] = acc2s[c] + v * v
                    acc = (accs[0] + accs[1]) + (accs[2] + accs[3])
                    acc2 = (acc2s[0] + acc2s[1]) + (acc2s[2] + acc2s[3])
                    r = jnp.sum(acc * acc - acc2) * 0.5
                    return jnp.where(lane == j, r, res)

                o_vmem[g, :] = lax.fori_loop(
                    0, _L, per_sample, jnp.zeros((_L,), jnp.float32)
                )

        pltpu.emit_pipeline(
            body,
            grid=(b // _BS,),
            in_specs=[pl.BlockSpec((_BS, f * d), lambda i: (i, 0))],
            out_specs=[pl.BlockSpec((_BS // _L, _L), lambda i: (i, 0))],
            core_axis_name=("core", "subcore"),
            dimension_semantics=(pltpu.PARALLEL,),
        )(x_hbm, o_hbm)

    return k(x2d)


def kernel(feature_emb):
    b, f, d = feature_emb.shape
    x2d = feature_emb.reshape(b, f * d)
    return _sc_pool(x2d, b, f, d).reshape(b, 1)


# speedup vs baseline: 1.9000x; 1.9000x over previous
import dataclasses
import functools

import jax
import jax.numpy as jnp
from jax import lax
from jax.experimental import pallas as pl
from jax.experimental.pallas import tpu as pltpu
from jax.experimental.pallas import tpu_sc as plsc

_L = 16
_COLS = 128
_FSEG = 25
_NSEG = 4
_NACC = _L + 1


@functools.partial(jax.jit, static_argnums=(1, 2, 3))
def _sc_pool_t(xt, b, f, d):
    mesh = plsc.VectorSubcoreMesh(core_axis_name="core", subcore_axis_name="subcore")
    cp = pltpu.CompilerParams()
    if "needs_layout_passes" in pltpu.CompilerParams.__dataclass_fields__:
        cp = dataclasses.replace(cp, needs_layout_passes=False)
    seg_rows = _FSEG * d
    n_lg = _COLS // _L

    @functools.partial(
        pl.kernel,
        out_type=jax.ShapeDtypeStruct((b,), jnp.float32),
        mesh=mesh,
        compiler_params=cp,
        scratch_types=[pltpu.VMEM((n_lg * _NACC * _L,), jnp.float32)],
    )
    def k(x_hbm, o_hbm, acc_ref):
        def body(x_vmem, o_vmem, acc):
            r = pl.program_id(1)
            first = r == 0

            @pl.loop(0, n_lg)
            def per_lane_group(g):
                base = g * (_NACC * _L)
                state = tuple(
                    jnp.where(first, 0.0, acc[pl.ds(base + t * _L, _L)])
                    for t in range(_NACC)
                )

                def fstep(fi, carry):
                    row = fi * d
                    ss = list(carry[:_L])
                    q = carry[_L]
                    for dd in range(d):
                        v = x_vmem[row + dd, pl.ds(g * _L, _L)]
                        ss[dd] = ss[dd] + v
                        q = q + v * v
                    return tuple(ss) + (q,)

                state = lax.fori_loop(0, _FSEG, fstep, state)
                for t in range(_NACC):
                    acc[pl.ds(base + t * _L, _L)] = state[t]
                tot = state[0] * state[0]
                for t in range(1, _L):
                    tot = tot + state[t] * state[t]
                o_vmem[pl.ds(g * _L, _L)] = (tot - state[_L]) * 0.5

        pltpu.emit_pipeline(
            body,
            grid=(b // _COLS, _NSEG),
            in_specs=[pl.BlockSpec((seg_rows, _COLS), lambda i, j: (j, i))],
            out_specs=[pl.BlockSpec((_COLS,), lambda i, j: (i,))],
            core_axis_name=("core", "subcore"),
            dimension_semantics=(pltpu.PARALLEL, pltpu.ARBITRARY),
        )(x_hbm, o_hbm, scratches=[acc_ref])

    return k(xt)


def kernel(feature_emb):
    b, f, d = feature_emb.shape
    xt = feature_emb.transpose(1, 2, 0).reshape(f * d, b)
    return _sc_pool_t(xt, b, f, d).reshape(b, 1)
